# baseline (speedup 1.0000x reference)
import jax
import jax.numpy as jnp
from jax.experimental import pallas as pl

_STRIDES = (8.0, 16.0, 32.0)
_WLOG = (6, 5, 4)
_NS = (4096, 1024, 256)
_C = 85
_ROWS = tuple(n * _C // 128 for n in _NS)
_RTOT = sum(_ROWS)
_NTOT = sum(_NS)


def _decode_level(v, stride, wlog):
    rows = v.shape[0]
    r = jax.lax.broadcasted_iota(jnp.int32, (rows, 128), 0)
    l = jax.lax.broadcasted_iota(jnp.int32, (rows, 128), 1)
    q = (r << 7) | l
    d = q // _C
    c = q - d * _C
    gx = (d & ((1 << wlog) - 1)).astype(jnp.float32)
    gy = (d >> wlog).astype(jnp.float32)
    g = jnp.where(c == 0, gx, gy)
    xy = (v + g) * stride
    wh = jnp.exp(v) * stride
    return jnp.where(c >= 4, v, jnp.where(c < 2, xy, wh))


def _decode_kernel(f0_ref, f1_ref, f2_ref, out_ref):
    off = 0
    for ref, stride, wlog, rows in zip(
        (f0_ref, f1_ref, f2_ref), _STRIDES, _WLOG, _ROWS
    ):
        out_ref[0, pl.ds(off, rows), :] = _decode_level(ref[0], stride, wlog)
        off += rows


@jax.jit
def kernel(feat0, feat1, feat2, targets):
    b = feat0.shape[0]
    f0 = feat0.reshape(b, _ROWS[0], 128)
    f1 = feat1.reshape(b, _ROWS[1], 128)
    f2 = feat2.reshape(b, _ROWS[2], 128)
    out = pl.pallas_call(
        _decode_kernel,
        grid=(b,),
        in_specs=[
            pl.BlockSpec((1, _ROWS[0], 128), lambda i: (i, 0, 0)),
            pl.BlockSpec((1, _ROWS[1], 128), lambda i: (i, 0, 0)),
            pl.BlockSpec((1, _ROWS[2], 128), lambda i: (i, 0, 0)),
        ],
        out_specs=pl.BlockSpec((1, _RTOT, 128), lambda i: (i, 0, 0)),
        out_shape=jax.ShapeDtypeStruct((b, _RTOT, 128), jnp.float32),
    )(f0, f1, f2)
    return out.reshape(b, _NTOT, _C)


# speedup vs baseline: 2.7735x; 2.7735x over previous
import jax
import jax.numpy as jnp
from jax.experimental import pallas as pl

_STRIDES = (8.0, 16.0, 32.0)
_WS = (64, 32, 16)
_NS = (4096, 1024, 256)
_OFFS = (0, 4096, 5120)
_NTOT = 5376
_C = 85


def _decode_level(v, stride, w):
    n = v.shape[0]
    p = jax.lax.broadcasted_iota(jnp.int32, (n, 1), 0)
    gx = (p & (w - 1)).astype(jnp.float32)
    gy = (p // w).astype(jnp.float32)
    c = jax.lax.broadcasted_iota(jnp.int32, (n, _C), 1)
    g = jnp.where(c == 0, gx, gy)
    xy = (v + g) * stride
    wh = jnp.exp(v) * stride
    return jnp.where(c < 2, xy, jnp.where(c < 4, wh, v))


def _decode_kernel(f0_ref, f1_ref, f2_ref, out_ref):
    for ref, stride, w, off, n in zip(
        (f0_ref, f1_ref, f2_ref), _STRIDES, _WS, _OFFS, _NS
    ):
        out_ref[0, pl.ds(off, n), :] = _decode_level(ref[0], stride, w)


@jax.jit
def kernel(feat0, feat1, feat2, targets):
    b = feat0.shape[0]
    f0 = feat0.reshape(b, _NS[0], _C)
    f1 = feat1.reshape(b, _NS[1], _C)
    f2 = feat2.reshape(b, _NS[2], _C)
    return pl.pallas_call(
        _decode_kernel,
        grid=(b,),
        in_specs=[
            pl.BlockSpec((1, _NS[0], _C), lambda i: (i, 0, 0)),
            pl.BlockSpec((1, _NS[1], _C), lambda i: (i, 0, 0)),
            pl.BlockSpec((1, _NS[2], _C), lambda i: (i, 0, 0)),
        ],
        out_specs=pl.BlockSpec((1, _NTOT, _C), lambda i: (i, 0, 0)),
        out_shape=jax.ShapeDtypeStruct((b, _NTOT, _C), jnp.float32),
    )(f0, f1, f2)
